# baseline (speedup 1.0000x reference)
import jax
import jax.numpy as jnp
from jax.experimental import pallas as pl
from jax.experimental.pallas import tpu as pltpu

_ROWS_PER_BLOCK = 64


def _topk_mask_kernel(k_ref, x_ref, o_ref):
    k = k_ref[0]
    x = x_ref[...]
    bits = jax.lax.bitcast_convert_type(x, jnp.int32) & jnp.int32(0x7FFFFFFF)

    r = x.shape[0]
    c = x.shape[1]

    lo = jnp.zeros((r, 1), jnp.int32)
    hi = jnp.full((r, 1), jnp.int32(0x7F800000))
    thr = jnp.zeros((r, 1), jnp.int32)
    done = jnp.zeros((r, 1), jnp.int32)

    def body1(_, carry):
        lo, hi, thr, done = carry
        mid = lo + ((hi - lo) >> 1)
        cnt = jnp.sum((bits >= mid).astype(jnp.int32), axis=1, keepdims=True)
        hit = cnt == k
        thr = jnp.where((done == 0) & hit, mid, thr)
        done = jnp.maximum(done, hit.astype(jnp.int32))
        pred = cnt >= k
        lo = jnp.where(pred, mid, lo)
        hi = jnp.where(pred, hi, mid)
        return lo, hi, thr, done

    carry = jax.lax.fori_loop(0, 19, body1, (lo, hi, thr, done))

    def maybe_more(n, carry):
        return jax.lax.cond(
            jnp.all(carry[3] == 1),
            lambda c: c,
            lambda c: jax.lax.fori_loop(0, n, body1, c),
            carry)

    for n in (4, 4, 4):
        carry = maybe_more(n, carry)
    lo, hi, thr, done = carry
    thr = jnp.where(done == 1, thr, lo)
    col = jax.lax.broadcasted_iota(jnp.int32, x.shape, 1)
    gt = bits > thr
    eq = bits == thr
    all_cols = jnp.full((r, 1), jnp.int32(c - 1))

    def tie_fix(_):
        need = k - jnp.sum(gt.astype(jnp.int32), axis=1, keepdims=True)

        def body2(_, carry):
            lo2, hi2 = carry
            mid = lo2 + ((hi2 - lo2) >> 1)
            cnt = jnp.sum((eq & (col <= mid)).astype(jnp.int32), axis=1,
                          keepdims=True)
            pred = cnt >= need
            hi2 = jnp.where(pred, mid, hi2)
            lo2 = jnp.where(pred, lo2, mid + 1)
            return lo2, hi2

        _, hi2 = jax.lax.fori_loop(
            0, max(c - 1, 1).bit_length(),
            body2,
            (jnp.zeros((r, 1), jnp.int32), all_cols),
        )
        return jnp.where(done == 1, all_cols, hi2)

    cutoff = jax.lax.cond(jnp.all(done == 1), lambda _: all_cols, tie_fix,
                          jnp.int32(0))
    keep = gt | (eq & (col <= cutoff))
    o_ref[...] = jnp.where(keep, x, jnp.zeros_like(x))


def kernel(scores, k):
    n, c = scores.shape
    r = _ROWS_PER_BLOCK
    k_arr = jnp.asarray(k, jnp.int32).reshape(1)
    return pl.pallas_call(
        _topk_mask_kernel,
        grid=(n // r,),
        in_specs=[
            pl.BlockSpec(memory_space=pltpu.SMEM),
            pl.BlockSpec((r, c), lambda i: (i, 0)),
        ],
        out_specs=pl.BlockSpec((r, c), lambda i: (i, 0)),
        out_shape=jax.ShapeDtypeStruct((n, c), scores.dtype),
    )(k_arr, scores)


# speedup vs baseline: 1.0234x; 1.0234x over previous
import jax
import jax.numpy as jnp
from jax.experimental import pallas as pl
from jax.experimental.pallas import tpu as pltpu

_ROWS_PER_BLOCK = 32


def _topk_mask_kernel(k_ref, x_ref, o_ref):
    k = k_ref[0]
    x = x_ref[...]
    bits = jax.lax.bitcast_convert_type(x, jnp.int32) & jnp.int32(0x7FFFFFFF)

    r = x.shape[0]
    c = x.shape[1]

    lo = jnp.zeros((r, 1), jnp.int32)
    hi = jnp.full((r, 1), jnp.int32(0x7F800000))
    thr = jnp.zeros((r, 1), jnp.int32)
    done = jnp.zeros((r, 1), jnp.int32)

    def body1(_, carry):
        lo, hi, thr, done = carry
        mid = lo + ((hi - lo) >> 1)
        cnt = jnp.sum((bits >= mid).astype(jnp.int32), axis=1, keepdims=True)
        hit = cnt == k
        thr = jnp.where((done == 0) & hit, mid, thr)
        done = jnp.maximum(done, hit.astype(jnp.int32))
        pred = cnt >= k
        lo = jnp.where(pred, mid, lo)
        hi = jnp.where(pred, hi, mid)
        return lo, hi, thr, done

    carry = jax.lax.fori_loop(0, 19, body1, (lo, hi, thr, done))

    def maybe_more(n, carry):
        return jax.lax.cond(
            jnp.all(carry[3] == 1),
            lambda c: c,
            lambda c: jax.lax.fori_loop(0, n, body1, c),
            carry)

    for n in (4, 4, 4):
        carry = maybe_more(n, carry)
    lo, hi, thr, done = carry
    thr = jnp.where(done == 1, thr, lo)

    def write_simple(_):
        o_ref[...] = jnp.where(bits >= thr, x, jnp.zeros_like(x))

    def write_tie(_):
        col = jax.lax.broadcasted_iota(jnp.int32, x.shape, 1)
        gt = bits > thr
        eq = bits == thr
        all_cols = jnp.full((r, 1), jnp.int32(c - 1))
        need = k - jnp.sum(gt.astype(jnp.int32), axis=1, keepdims=True)

        def body2(_, carry):
            lo2, hi2 = carry
            mid = lo2 + ((hi2 - lo2) >> 1)
            cnt = jnp.sum((eq & (col <= mid)).astype(jnp.int32), axis=1,
                          keepdims=True)
            pred = cnt >= need
            hi2 = jnp.where(pred, mid, hi2)
            lo2 = jnp.where(pred, lo2, mid + 1)
            return lo2, hi2

        _, hi2 = jax.lax.fori_loop(
            0, max(c - 1, 1).bit_length(),
            body2,
            (jnp.zeros((r, 1), jnp.int32), all_cols),
        )
        cutoff = jnp.where(done == 1, all_cols, hi2)
        keep = gt | (eq & (col <= cutoff))
        o_ref[...] = jnp.where(keep, x, jnp.zeros_like(x))

    jax.lax.cond(jnp.all(done == 1), write_simple, write_tie, jnp.int32(0))


def kernel(scores, k):
    n, c = scores.shape
    r = _ROWS_PER_BLOCK
    k_arr = jnp.asarray(k, jnp.int32).reshape(1)
    return pl.pallas_call(
        _topk_mask_kernel,
        grid=(n // r,),
        in_specs=[
            pl.BlockSpec(memory_space=pltpu.SMEM),
            pl.BlockSpec((r, c), lambda i: (i, 0)),
        ],
        out_specs=pl.BlockSpec((r, c), lambda i: (i, 0)),
        out_shape=jax.ShapeDtypeStruct((n, c), scores.dtype),
    )(k_arr, scores)
